# baseline (speedup 1.0000x reference)
import functools

import jax
import jax.numpy as jnp
import numpy as np
from jax import lax
from jax.experimental import pallas as pl
from jax.experimental.pallas import tpu as pltpu
from jax.experimental.pallas import tpu_sc as plsc

_G = 4194304
_NW = 32
_CH = 2048
_GPW = _G // _NW
_NCH = _GPW // _CH

_LN2 = np.float32(0.6931471805599453)
_SQRT2 = np.float32(1.4142135623730951)
_C = tuple(np.float32(c) for c in (
    0.9999728288274139, -0.49938652694242347, 0.33593280906047096,
    -0.27203310709725076, 0.18102717325886228))


def _vln(x):
    bits = lax.bitcast_convert_type(x, jnp.int32)
    e = (bits >> 23) - 127
    m = lax.bitcast_convert_type(
        (bits & jnp.int32(0x007FFFFF)) | jnp.int32(0x3F800000), jnp.float32)
    big = m > _SQRT2
    m = jnp.where(big, m * np.float32(0.5), m)
    ef = (e + jnp.where(big, jnp.int32(1), jnp.int32(0))).astype(jnp.float32)
    f = m - np.float32(1.0)
    p = _C[4]
    for c in (_C[3], _C[2], _C[1], _C[0]):
        p = p * f + c
    return ef * _LN2 + f * p


_mesh = plsc.VectorSubcoreMesh(core_axis_name="c", subcore_axis_name="s")


@functools.partial(
    pl.kernel,
    mesh=_mesh,
    compiler_params=pltpu.CompilerParams(needs_layout_passes=False),
    out_type=jax.ShapeDtypeStruct((_G * 4,), jnp.float32),
    scratch_types=[
        pltpu.VMEM((_CH * 6,), jnp.float32),
        pltpu.VMEM((_CH * 6,), jnp.float32),
        pltpu.VMEM((_CH * 4,), jnp.float32),
    ],
)
def _sc_forward(gate_hbm, u_hbm, out_hbm, gbuf, ubuf, obuf):
    wid = lax.axis_index("c") * 16 + lax.axis_index("s")
    base_g = wid * _GPW
    iota = lax.broadcasted_iota(jnp.int32, (16,), 0)
    idx6 = iota * 6
    idx4 = iota * 4

    def chunk(c, carry):
        g0 = base_g + c * _CH
        pltpu.sync_copy(gate_hbm.at[pl.ds(g0 * 6, _CH * 6)], gbuf)
        pltpu.sync_copy(u_hbm.at[pl.ds(g0 * 6, _CH * 6)], ubuf)

        def it(i, icarry):
            b6 = i * 96
            xs = [plsc.load_gather(gbuf, [idx6 + (b6 + k)]) for k in range(6)]
            us = [plsc.load_gather(ubuf, [idx6 + (b6 + k)]) for k in range(6)]
            zs = []
            for k in range(6):
                t = -_vln(us[k])
                gmb = -_vln(t)
                zs.append(xs[k] * np.float32(1000.0 / 3.0)
                          + gmb * np.float32(1.0 / 3.0))
            zmax = zs[0]
            for k in range(1, 6):
                zmax = jnp.maximum(zmax, zs[k])
            es = [jnp.exp(z - zmax) for z in zs]
            r = np.float32(1.0) / (es[0] + es[1] + es[2] + es[3] + es[4] + es[5])
            outs = (
                (es[0] + es[1] + es[2]) * r,
                (es[0] + es[3] + es[4]) * r,
                (es[1] + es[3] + es[5]) * r,
                (es[2] + es[4] + es[5]) * r,
            )
            b4 = i * 64
            for j in range(4):
                plsc.store_scatter(obuf, [idx4 + (b4 + j)], outs[j])
            return icarry

        lax.fori_loop(0, _CH // 16, it, 0)
        pltpu.sync_copy(obuf, out_hbm.at[pl.ds(g0 * 4, _CH * 4)])
        return carry

    lax.fori_loop(0, _NCH, chunk, 0)


def kernel(gate, mask_options, u):
    del mask_options
    gf = gate.reshape(_G * 6)
    uf = u.reshape(_G * 6)
    of = _sc_forward(gf, uf)
    return of.reshape(4096, 4096)
